# baseline (speedup 1.0000x reference)
import functools

import jax
import jax.numpy as jnp
from jax import lax
from jax.experimental import pallas as pl
from jax.experimental.pallas import tpu as pltpu
from jax.experimental.pallas import tpu_sc as plsc

B = 16384
F = 32
M = 128
NC = 2
NS = 16
NW = NC * NS
BPW = B // NW
CH = 128
NCH = BPW // CH

U = 100000
RU = 128
PR = RU // 4
NUNITS = U // RU
TAIL = U - NUNITS * RU
KMAX = (NUNITS + 1 + NW - 1) // NW
UP = U // 4

_MESH = dict(core_axis_name="c", subcore_axis_name="s",
             num_cores=NC, num_subcores=NS)


def _worker_base():
    wid = lax.axis_index("s") * NC + lax.axis_index("c")
    return wid, wid * BPW


def _repack_unit(table, packed, rin, rout, u, nrows):
    pltpu.sync_copy(table.at[pl.ds(u * RU, nrows)], rin.at[pl.ds(0, nrows)])
    for r in range(nrows):
        for c in (0, 16):
            rout[r // 4, pl.ds((r % 4) * F + c, 16)] = rin[r, pl.ds(c, 16)]
    pltpu.sync_copy(rout.at[pl.ds(0, nrows // 4)],
                    packed.at[pl.ds(u * PR, nrows // 4)])


def _repack_table(table, packed, rin, rout, wid):
    def step(k, _):
        u = k * NW + wid

        @pl.when(u < NUNITS)
        def _full():
            _repack_unit(table, packed, rin, rout, u, RU)

        @pl.when(u == NUNITS)
        def _tail():
            _repack_unit(table, packed, rin, rout, u, TAIL)

        return _
    lax.fori_loop(0, KMAX, step, None)


def _sc_a_body(user3d, item3d, tum, tim, tug, tig,
               oum, oim, pug, pig,
               idx_u, idx_i, mb, rin, rout, smb):
    wid, base = _worker_base()
    pltpu.sync_copy(user3d.at[wid], idx_u)
    pltpu.sync_copy(item3d.at[wid], idx_i)
    cu = [pltpu.async_copy(tum.at[idx_u.at[j]], mb.at[j], smb.at[j])
          for j in range(NCH)]
    _repack_table(tug, pug, rin, rout, wid)
    for j in range(NCH):
        cu[j].wait()
        pltpu.sync_copy(mb.at[j], oum.at[pl.ds(base + j * CH, CH)])
    ci = [pltpu.async_copy(tim.at[idx_i.at[j]], mb.at[j], smb.at[j])
          for j in range(NCH)]
    _repack_table(tig, pig, rin, rout, wid)
    for j in range(NCH):
        ci[j].wait()
        pltpu.sync_copy(mb.at[j], oim.at[pl.ds(base + j * CH, CH)])


def _sc_b_body(pu3d, pi3d, pug, pig, ogu, ogi,
               idx_u, idx_i, gb, sgb):
    wid, base = _worker_base()
    pltpu.sync_copy(pu3d.at[wid], idx_u)
    pltpu.sync_copy(pi3d.at[wid], idx_i)
    cu = [pltpu.async_copy(pug.at[idx_u.at[j]], gb.at[j], sgb.at[j])
          for j in range(NCH)]
    for j in range(NCH):
        cu[j].wait()
        pltpu.sync_copy(gb.at[j], ogu.at[pl.ds(base + j * CH, CH)])
    ci = [pltpu.async_copy(pig.at[idx_i.at[j]], gb.at[j], sgb.at[j])
          for j in range(NCH)]
    for j in range(NCH):
        ci[j].wait()
        pltpu.sync_copy(gb.at[j], ogi.at[pl.ds(base + j * CH, CH)])


@jax.jit
def _sc_gather(user3d, item3d, pu3d, pi3d, tug, tig, tum, tim):
    f32 = jnp.float32
    um, im, pug, pig = pl.kernel(
        _sc_a_body,
        out_type=(
            jax.ShapeDtypeStruct((B, M), f32),
            jax.ShapeDtypeStruct((B, M), f32),
            jax.ShapeDtypeStruct((UP, M), f32),
            jax.ShapeDtypeStruct((UP, M), f32),
        ),
        mesh=plsc.VectorSubcoreMesh(**_MESH),
        scratch_types=(
            pltpu.VMEM((NCH, CH), jnp.int32),
            pltpu.VMEM((NCH, CH), jnp.int32),
            pltpu.VMEM((NCH, CH, M), f32),
            pltpu.VMEM((RU, F), f32),
            pltpu.VMEM((PR, M), f32),
            pltpu.SemaphoreType.DMA((NCH,)),
        ),
    )(user3d, item3d, tum, tim, tug, tig)
    gu, gi = pl.kernel(
        _sc_b_body,
        out_type=(
            jax.ShapeDtypeStruct((B, M), f32),
            jax.ShapeDtypeStruct((B, M), f32),
        ),
        mesh=plsc.VectorSubcoreMesh(**_MESH),
        scratch_types=(
            pltpu.VMEM((NCH, CH), jnp.int32),
            pltpu.VMEM((NCH, CH), jnp.int32),
            pltpu.VMEM((NCH, CH, M), f32),
            pltpu.SemaphoreType.DMA((NCH,)),
        ),
    )(pu3d, pi3d, pug, pig)
    return um, im, gu, gi


def _tc_body(gu, gi, lu, li, um, im, w1u, w1i, b1, w2, b2, w3, b3,
             wpg, wph, bp, out):
    h = (jnp.dot(um[...], w1u[...]) + jnp.dot(im[...], w1i[...]) + b1[...])
    h = jnp.maximum(h, 0.0)
    h = jnp.maximum(jnp.dot(h, w2[...]) + b2[...], 0.0)
    h = jnp.maximum(jnp.dot(h, w3[...]) + b3[...], 0.0)
    luv = lu[...][:, None]
    liv = li[...][:, None]
    ug = jnp.zeros_like(gu[...][:, :F])
    ig = jnp.zeros_like(ug)
    for k in range(4):
        ug = ug + jnp.where(luv == k, gu[...][:, k * F:(k + 1) * F], 0.0)
        ig = ig + jnp.where(liv == k, gi[...][:, k * F:(k + 1) * F], 0.0)
    g = ug * ig
    pred = (jnp.sum(g * wpg[...], axis=1)
            + jnp.sum(h * wph[...], axis=1) + bp[0, 0])
    out[...] = pred


@functools.partial(jax.jit, static_argnames=("blk",))
def _tc_mlp(gu, gi, lu, li, um, im, w1u, w1i, b1, w2, b2, w3, b3,
            wpg, wph, bp, blk=2048):
    grid = (B // blk,)
    full = lambda shape: pl.BlockSpec(shape, lambda i: (0, 0))
    row = pl.BlockSpec((blk,), lambda i: (i,))
    return pl.pallas_call(
        _tc_body,
        grid=grid,
        in_specs=[
            pl.BlockSpec((blk, M), lambda i: (i, 0)),
            pl.BlockSpec((blk, M), lambda i: (i, 0)),
            row, row,
            pl.BlockSpec((blk, M), lambda i: (i, 0)),
            pl.BlockSpec((blk, M), lambda i: (i, 0)),
            full((M, M)), full((M, M)), full((1, M)),
            full((M, M // 2)), full((1, M // 2)),
            full((M // 2, F)), full((1, F)),
            full((1, F)), full((1, F)), full((1, 1)),
        ],
        out_specs=row,
        out_shape=jax.ShapeDtypeStruct((B,), jnp.float32),
    )(gu, gi, lu, li, um, im, w1u, w1i, b1, w2, b2, w3, b3, wpg, wph, bp)


def kernel(user, item, embed_user_GMF, embed_item_GMF, embed_user_MLP,
           embed_item_MLP, W1, b1, W2, b2, W3, b3, Wp, bp):
    user = user.astype(jnp.int32)
    item = item.astype(jnp.int32)
    user3d = user.reshape(NW, NCH, CH)
    item3d = item.reshape(NW, NCH, CH)
    pu3d = (user // 4).reshape(NW, NCH, CH)
    pi3d = (item // 4).reshape(NW, NCH, CH)
    um, im, gu, gi = _sc_gather(user3d, item3d, pu3d, pi3d,
                                embed_user_GMF, embed_item_GMF,
                                embed_user_MLP, embed_item_MLP)
    pred = _tc_mlp(gu, gi, user % 4, item % 4, um, im,
                   W1[:M], W1[M:], b1.reshape(1, M),
                   W2, b2.reshape(1, M // 2),
                   W3, b3.reshape(1, F),
                   Wp[:F, 0].reshape(1, F), Wp[F:, 0].reshape(1, F),
                   bp.reshape(1, 1))
    return pred


# speedup vs baseline: 1.5358x; 1.5358x over previous
import functools

import jax
import jax.numpy as jnp
from jax import lax
from jax.experimental import pallas as pl
from jax.experimental.pallas import tpu as pltpu
from jax.experimental.pallas import tpu_sc as plsc

B = 16384
F = 32
M = 128
U = 100000
NC = 2
NS = 16
NW = NC * NS
BPW = B // NW
CH = 128
NCH = BPW // CH
NG = CH // 16

_MESH = dict(core_axis_name="c", subcore_axis_name="s",
             num_cores=NC, num_subcores=NS)


def _worker_base():
    wid = lax.axis_index("s") * NC + lax.axis_index("c")
    return wid, wid * BPW


def _sc_mlp_body(user3d, item3d, tum, tim, oum, oim,
                 idx_u, idx_i, mb, smb):
    wid, base = _worker_base()
    pltpu.sync_copy(user3d.at[wid], idx_u)
    pltpu.sync_copy(item3d.at[wid], idx_i)
    cu = [pltpu.async_copy(tum.at[idx_u.at[j]], mb.at[j], smb.at[j])
          for j in range(NCH)]
    for j in range(NCH):
        cu[j].wait()
        pltpu.sync_copy(mb.at[j], oum.at[pl.ds(base + j * CH, CH)])
    ci = [pltpu.async_copy(tim.at[idx_i.at[j]], mb.at[j], smb.at[j])
          for j in range(NCH)]
    for j in range(NCH):
        ci[j].wait()
        pltpu.sync_copy(mb.at[j], oim.at[pl.ds(base + j * CH, CH)])


def _gmf_table(tflat, out, idx, wid, idxd, gw, sg):
    for j in range(NCH):
        for c in range(F):
            for g in range(NG):
                idxd[c, pl.ds(g * 16, 16)] = (
                    idx[j, pl.ds(g * 16, 16)] + c * U)
        for half in range(2):
            cs = [pltpu.async_copy(tflat.at[idxd.at[c]], gw.at[c, j], sg)
                  for c in range(half * 16, half * 16 + 16)]
            for c in cs:
                c.wait()
    pltpu.sync_copy(gw, out.at[wid])


def _sc_gmf_body(user3d, item3d, tugf, tigf, oug, oig,
                 idx_u, idx_i, idxd, gw, sg):
    wid, _ = _worker_base()
    pltpu.sync_copy(user3d.at[wid], idx_u)
    pltpu.sync_copy(item3d.at[wid], idx_i)
    _gmf_table(tugf, oug, idx_u, wid, idxd, gw, sg)
    _gmf_table(tigf, oig, idx_i, wid, idxd, gw, sg)


@jax.jit
def _sc_gather(user3d, item3d, tugf, tigf, tum, tim):
    f32 = jnp.float32
    um, im = pl.kernel(
        _sc_mlp_body,
        out_type=(
            jax.ShapeDtypeStruct((B, M), f32),
            jax.ShapeDtypeStruct((B, M), f32),
        ),
        mesh=plsc.VectorSubcoreMesh(**_MESH),
        scratch_types=(
            pltpu.VMEM((NCH, CH), jnp.int32),
            pltpu.VMEM((NCH, CH), jnp.int32),
            pltpu.VMEM((NCH, CH, M), f32),
            pltpu.SemaphoreType.DMA((NCH,)),
        ),
    )(user3d, item3d, tum, tim)
    ug, ig = pl.kernel(
        _sc_gmf_body,
        out_type=(
            jax.ShapeDtypeStruct((NW, F, NCH, CH), f32),
            jax.ShapeDtypeStruct((NW, F, NCH, CH), f32),
        ),
        mesh=plsc.VectorSubcoreMesh(**_MESH),
        compiler_params=pltpu.CompilerParams(use_tc_tiling_on_sc=False),
        scratch_types=(
            pltpu.VMEM((NCH, CH), jnp.int32),
            pltpu.VMEM((NCH, CH), jnp.int32),
            pltpu.VMEM((F, CH), jnp.int32),
            pltpu.VMEM((F, NCH, CH), f32),
            pltpu.SemaphoreType.DMA,
        ),
    )(user3d, item3d, tugf, tigf)
    return um, im, ug, ig


def _tc_mlp_body(um, im, w1u, w1i, b1, w2, b2, w3, b3, wph, bp, out):
    h = (jnp.dot(um[...], w1u[...]) + jnp.dot(im[...], w1i[...]) + b1[...])
    h = jnp.maximum(h, 0.0)
    h = jnp.maximum(jnp.dot(h, w2[...]) + b2[...], 0.0)
    h = jnp.maximum(jnp.dot(h, w3[...]) + b3[...], 0.0)
    out[...] = jnp.sum(h * wph[...], axis=1) + bp[0, 0]


@functools.partial(jax.jit, static_argnames=("blk",))
def _tc_mlp(um, im, w1u, w1i, b1, w2, b2, w3, b3, wph, bp, blk=2048):
    grid = (B // blk,)
    full = lambda shape: pl.BlockSpec(shape, lambda i: (0, 0))
    return pl.pallas_call(
        _tc_mlp_body,
        grid=grid,
        in_specs=[
            pl.BlockSpec((blk, M), lambda i: (i, 0)),
            pl.BlockSpec((blk, M), lambda i: (i, 0)),
            full((M, M)), full((M, M)), full((1, M)),
            full((M, M // 2)), full((1, M // 2)),
            full((M // 2, F)), full((1, F)),
            full((1, F)), full((1, 1)),
        ],
        out_specs=pl.BlockSpec((blk,), lambda i: (i,)),
        out_shape=jax.ShapeDtypeStruct((B,), jnp.float32),
    )(um, im, w1u, w1i, b1, w2, b2, w3, b3, wph, bp)


def _tc_gmf_body(ug, ig, p1, wpg, out):
    g = ug[...] * ig[...] * wpg[...]
    out[...] = jnp.sum(g, axis=1) + p1[...]


@functools.partial(jax.jit, static_argnames=("bw",))
def _tc_gmf(ug, ig, p1, wpg, bw=4):
    grid = (NW // bw,)
    blk4 = pl.BlockSpec((bw, F, NCH, CH), lambda i: (i, 0, 0, 0))
    return pl.pallas_call(
        _tc_gmf_body,
        grid=grid,
        in_specs=[
            blk4, blk4,
            pl.BlockSpec((bw, NCH, CH), lambda i: (i, 0, 0)),
            pl.BlockSpec((1, F, 1, 1), lambda i: (0, 0, 0, 0)),
        ],
        out_specs=pl.BlockSpec((bw, NCH, CH), lambda i: (i, 0, 0)),
        out_shape=jax.ShapeDtypeStruct((NW, NCH, CH), jnp.float32),
    )(ug, ig, p1, wpg)


def kernel(user, item, embed_user_GMF, embed_item_GMF, embed_user_MLP,
           embed_item_MLP, W1, b1, W2, b2, W3, b3, Wp, bp):
    user = user.astype(jnp.int32)
    item = item.astype(jnp.int32)
    user3d = user.reshape(NW, NCH, CH)
    item3d = item.reshape(NW, NCH, CH)
    tugf = embed_user_GMF.T.reshape(-1)
    tigf = embed_item_GMF.T.reshape(-1)
    um, im, ug, ig = _sc_gather(user3d, item3d, tugf, tigf,
                                embed_user_MLP, embed_item_MLP)
    p1 = _tc_mlp(um, im,
                 W1[:M], W1[M:], b1.reshape(1, M),
                 W2, b2.reshape(1, M // 2),
                 W3, b3.reshape(1, F),
                 Wp[F:, 0].reshape(1, F), bp.reshape(1, 1))
    pred = _tc_gmf(ug, ig, p1.reshape(NW, NCH, CH),
                   Wp[:F, 0].reshape(1, F, 1, 1))
    return pred.reshape(B)


# speedup vs baseline: 1.6136x; 1.0506x over previous
import functools

import jax
import jax.numpy as jnp
from jax import lax
from jax.experimental import pallas as pl
from jax.experimental.pallas import tpu as pltpu
from jax.experimental.pallas import tpu_sc as plsc

B = 16384
F = 32
M = 128
U = 100000
NC = 2
NS = 16
NW = NC * NS
BPW = B // NW
CH = 128
NCH = BPW // CH
NG = CH // 16

_MESH = dict(core_axis_name="c", subcore_axis_name="s",
             num_cores=NC, num_subcores=NS)


def _worker_base():
    wid = lax.axis_index("s") * NC + lax.axis_index("c")
    return wid, wid * BPW


def _sc_mlp_body(user3d, item3d, tum, tim, oum, oim,
                 idx_u, idx_i, mb, smb):
    wid, base = _worker_base()
    pltpu.sync_copy(user3d.at[wid], idx_u)
    pltpu.sync_copy(item3d.at[wid], idx_i)
    cu = [pltpu.async_copy(tum.at[idx_u.at[j]], mb.at[j], smb.at[j])
          for j in range(NCH)]
    for j in range(NCH):
        cu[j].wait()
        pltpu.sync_copy(mb.at[j], oum.at[pl.ds(base + j * CH, CH)])
    ci = [pltpu.async_copy(tim.at[idx_i.at[j]], mb.at[j], smb.at[j])
          for j in range(NCH)]
    for j in range(NCH):
        ci[j].wait()
        pltpu.sync_copy(mb.at[j], oim.at[pl.ds(base + j * CH, CH)])


def _gmf_table(tflat, out, idx, wid, idxd, gw, sg):
    for j in range(NCH):
        for c in range(F):
            for g in range(NG):
                idxd[c, pl.ds(g * 16, 16)] = (
                    idx[j, pl.ds(g * 16, 16)] + c * U)
        for half in range(2):
            cs = [pltpu.async_copy(tflat.at[idxd.at[c]], gw.at[c, j], sg)
                  for c in range(half * 16, half * 16 + 16)]
            for c in cs:
                c.wait()
    pltpu.sync_copy(gw, out.at[wid])


def _sc_gmf_body(idx3d, tflat, out, idx, idxd, gw, sg):
    wid, _ = _worker_base()
    pltpu.sync_copy(idx3d.at[wid], idx)
    _gmf_table(tflat, out, idx, wid, idxd, gw, sg)


@jax.jit
def _sc_gather(user3d, item3d, tugf, tigf, tum, tim):
    f32 = jnp.float32
    um, im = pl.kernel(
        _sc_mlp_body,
        out_type=(
            jax.ShapeDtypeStruct((B, M), f32),
            jax.ShapeDtypeStruct((B, M), f32),
        ),
        mesh=plsc.VectorSubcoreMesh(**_MESH),
        scratch_types=(
            pltpu.VMEM((NCH, CH), jnp.int32),
            pltpu.VMEM((NCH, CH), jnp.int32),
            pltpu.VMEM((NCH, CH, M), f32),
            pltpu.SemaphoreType.DMA((NCH,)),
        ),
    )(user3d, item3d, tum, tim)
    gmf_call = functools.partial(
        pl.kernel,
        _sc_gmf_body,
        out_type=jax.ShapeDtypeStruct((NW, F, NCH, CH), f32),
        mesh=plsc.VectorSubcoreMesh(**_MESH),
        compiler_params=pltpu.CompilerParams(use_tc_tiling_on_sc=False),
        scratch_types=(
            pltpu.VMEM((NCH, CH), jnp.int32),
            pltpu.VMEM((F, CH), jnp.int32),
            pltpu.VMEM((F, NCH, CH), f32),
            pltpu.SemaphoreType.DMA,
        ),
    )
    ug = gmf_call()(user3d, tugf)
    ig = gmf_call()(item3d, tigf)
    return um, im, ug, ig


def _tc_mlp_body(um, im, w1u, w1i, b1, w2, b2, w3, b3, wph, bp, out):
    h = (jnp.dot(um[...], w1u[...]) + jnp.dot(im[...], w1i[...]) + b1[...])
    h = jnp.maximum(h, 0.0)
    h = jnp.maximum(jnp.dot(h, w2[...]) + b2[...], 0.0)
    h = jnp.maximum(jnp.dot(h, w3[...]) + b3[...], 0.0)
    out[...] = jnp.sum(h * wph[...], axis=1) + bp[0, 0]


@functools.partial(jax.jit, static_argnames=("blk",))
def _tc_mlp(um, im, w1u, w1i, b1, w2, b2, w3, b3, wph, bp, blk=2048):
    grid = (B // blk,)
    full = lambda shape: pl.BlockSpec(shape, lambda i: (0, 0))
    return pl.pallas_call(
        _tc_mlp_body,
        grid=grid,
        in_specs=[
            pl.BlockSpec((blk, M), lambda i: (i, 0)),
            pl.BlockSpec((blk, M), lambda i: (i, 0)),
            full((M, M)), full((M, M)), full((1, M)),
            full((M, M // 2)), full((1, M // 2)),
            full((M // 2, F)), full((1, F)),
            full((1, F)), full((1, 1)),
        ],
        out_specs=pl.BlockSpec((blk,), lambda i: (i,)),
        out_shape=jax.ShapeDtypeStruct((B,), jnp.float32),
    )(um, im, w1u, w1i, b1, w2, b2, w3, b3, wph, bp)


def _tc_gmf_body(ug, ig, p1, wpg, out):
    g = ug[...] * ig[...] * wpg[...]
    out[...] = jnp.sum(g, axis=1) + p1[...]


@functools.partial(jax.jit, static_argnames=("bw",))
def _tc_gmf(ug, ig, p1, wpg, bw=4):
    grid = (NW // bw,)
    blk4 = pl.BlockSpec((bw, F, NCH, CH), lambda i: (i, 0, 0, 0))
    return pl.pallas_call(
        _tc_gmf_body,
        grid=grid,
        in_specs=[
            blk4, blk4,
            pl.BlockSpec((bw, NCH, CH), lambda i: (i, 0, 0)),
            pl.BlockSpec((1, F, 1, 1), lambda i: (0, 0, 0, 0)),
        ],
        out_specs=pl.BlockSpec((bw, NCH, CH), lambda i: (i, 0, 0)),
        out_shape=jax.ShapeDtypeStruct((NW, NCH, CH), jnp.float32),
    )(ug, ig, p1, wpg)


def kernel(user, item, embed_user_GMF, embed_item_GMF, embed_user_MLP,
           embed_item_MLP, W1, b1, W2, b2, W3, b3, Wp, bp):
    user = user.astype(jnp.int32)
    item = item.astype(jnp.int32)
    user3d = user.reshape(NW, NCH, CH)
    item3d = item.reshape(NW, NCH, CH)
    tugf = embed_user_GMF.T.reshape(-1)
    tigf = embed_item_GMF.T.reshape(-1)
    um, im, ug, ig = _sc_gather(user3d, item3d, tugf, tigf,
                                embed_user_MLP, embed_item_MLP)
    p1 = _tc_mlp(um, im,
                 W1[:M], W1[M:], b1.reshape(1, M),
                 W2, b2.reshape(1, M // 2),
                 W3, b3.reshape(1, F),
                 Wp[F:, 0].reshape(1, F), bp.reshape(1, 1))
    pred = _tc_gmf(ug, ig, p1.reshape(NW, NCH, CH),
                   Wp[:F, 0].reshape(1, F, 1, 1))
    return pred.reshape(B)


# speedup vs baseline: 1.7523x; 1.0860x over previous
import functools

import jax
import jax.numpy as jnp
from jax import lax
from jax.experimental import pallas as pl
from jax.experimental.pallas import tpu as pltpu
from jax.experimental.pallas import tpu_sc as plsc

B = 16384
F = 32
M = 128
U = 100000
NC = 2
NS = 16
NW = NC * NS
BPW = B // NW
CH = 128
NCH = BPW // CH
NG = CH // 16

_MESH = dict(core_axis_name="c", subcore_axis_name="s",
             num_cores=NC, num_subcores=NS)


def _worker_base():
    wid = lax.axis_index("s") * NC + lax.axis_index("c")
    return wid, wid * BPW


def _sc_mlp_body(user3d, item3d, tum, tim, oum, oim,
                 idx_u, idx_i, mb, smb):
    wid, base = _worker_base()
    pltpu.sync_copy(user3d.at[wid], idx_u)
    pltpu.sync_copy(item3d.at[wid], idx_i)
    cu = [pltpu.async_copy(tum.at[idx_u.at[j]], mb.at[j], smb.at[j])
          for j in range(NCH)]
    for j in range(NCH):
        cu[j].wait()
        pltpu.sync_copy(mb.at[j], oum.at[pl.ds(base + j * CH, CH)])
    ci = [pltpu.async_copy(tim.at[idx_i.at[j]], mb.at[j], smb.at[j])
          for j in range(NCH)]
    for j in range(NCH):
        ci[j].wait()
        pltpu.sync_copy(mb.at[j], oim.at[pl.ds(base + j * CH, CH)])


def _gmf_table(tflat, out, idx, wid, idxd2, gw, sg):
    descs = [None] * NCH
    for j in range(NCH):
        if j >= 2:
            for c in descs[j - 2]:
                c.wait()
        for c in range(F):
            for g in range(NG):
                idxd2[j % 2, c, pl.ds(g * 16, 16)] = (
                    idx[j, pl.ds(g * 16, 16)] + c * U)
        descs[j] = [pltpu.async_copy(tflat.at[idxd2.at[j % 2, c]],
                                     gw.at[c, j], sg.at[j % 2])
                    for c in range(F)]
    for j in range(NCH - 2, NCH):
        for c in descs[j]:
            c.wait()
    pltpu.sync_copy(gw, out.at[wid])


def _sc_gmf_body(idx3d, tflat, out, idx, idxd2, gw, sg):
    wid, _ = _worker_base()
    pltpu.sync_copy(idx3d.at[wid], idx)
    _gmf_table(tflat, out, idx, wid, idxd2, gw, sg)


@jax.jit
def _sc_gather(user3d, item3d, tugf, tigf, tum, tim):
    f32 = jnp.float32
    um, im = pl.kernel(
        _sc_mlp_body,
        out_type=(
            jax.ShapeDtypeStruct((B, M), f32),
            jax.ShapeDtypeStruct((B, M), f32),
        ),
        mesh=plsc.VectorSubcoreMesh(**_MESH),
        scratch_types=(
            pltpu.VMEM((NCH, CH), jnp.int32),
            pltpu.VMEM((NCH, CH), jnp.int32),
            pltpu.VMEM((NCH, CH, M), f32),
            pltpu.SemaphoreType.DMA((NCH,)),
        ),
    )(user3d, item3d, tum, tim)
    gmf_call = functools.partial(
        pl.kernel,
        _sc_gmf_body,
        out_type=jax.ShapeDtypeStruct((NW, F, NCH, CH), f32),
        mesh=plsc.VectorSubcoreMesh(**_MESH),
        compiler_params=pltpu.CompilerParams(use_tc_tiling_on_sc=False),
        scratch_types=(
            pltpu.VMEM((NCH, CH), jnp.int32),
            pltpu.VMEM((2, F, CH), jnp.int32),
            pltpu.VMEM((F, NCH, CH), f32),
            pltpu.SemaphoreType.DMA((2,)),
        ),
    )
    ug = gmf_call()(user3d, tugf)
    ig = gmf_call()(item3d, tigf)
    return um, im, ug, ig


def _tc_mlp_body(um, im, w1u, w1i, b1, w2, b2, w3, b3, wph, bp, out):
    h = (jnp.dot(um[...], w1u[...]) + jnp.dot(im[...], w1i[...]) + b1[...])
    h = jnp.maximum(h, 0.0)
    h = jnp.maximum(jnp.dot(h, w2[...]) + b2[...], 0.0)
    h = jnp.maximum(jnp.dot(h, w3[...]) + b3[...], 0.0)
    out[...] = jnp.sum(h * wph[...], axis=1) + bp[0, 0]


@functools.partial(jax.jit, static_argnames=("blk",))
def _tc_mlp(um, im, w1u, w1i, b1, w2, b2, w3, b3, wph, bp, blk=2048):
    grid = (B // blk,)
    full = lambda shape: pl.BlockSpec(shape, lambda i: (0, 0))
    return pl.pallas_call(
        _tc_mlp_body,
        grid=grid,
        in_specs=[
            pl.BlockSpec((blk, M), lambda i: (i, 0)),
            pl.BlockSpec((blk, M), lambda i: (i, 0)),
            full((M, M)), full((M, M)), full((1, M)),
            full((M, M // 2)), full((1, M // 2)),
            full((M // 2, F)), full((1, F)),
            full((1, F)), full((1, 1)),
        ],
        out_specs=pl.BlockSpec((blk,), lambda i: (i,)),
        out_shape=jax.ShapeDtypeStruct((B,), jnp.float32),
    )(um, im, w1u, w1i, b1, w2, b2, w3, b3, wph, bp)


def _tc_gmf_body(ug, ig, p1, wpg, out):
    g = ug[...] * ig[...] * wpg[...]
    out[...] = jnp.sum(g, axis=1) + p1[...]


@functools.partial(jax.jit, static_argnames=("bw",))
def _tc_gmf(ug, ig, p1, wpg, bw=4):
    grid = (NW // bw,)
    blk4 = pl.BlockSpec((bw, F, NCH, CH), lambda i: (i, 0, 0, 0))
    return pl.pallas_call(
        _tc_gmf_body,
        grid=grid,
        in_specs=[
            blk4, blk4,
            pl.BlockSpec((bw, NCH, CH), lambda i: (i, 0, 0)),
            pl.BlockSpec((1, F, 1, 1), lambda i: (0, 0, 0, 0)),
        ],
        out_specs=pl.BlockSpec((bw, NCH, CH), lambda i: (i, 0, 0)),
        out_shape=jax.ShapeDtypeStruct((NW, NCH, CH), jnp.float32),
    )(ug, ig, p1, wpg)


def kernel(user, item, embed_user_GMF, embed_item_GMF, embed_user_MLP,
           embed_item_MLP, W1, b1, W2, b2, W3, b3, Wp, bp):
    user = user.astype(jnp.int32)
    item = item.astype(jnp.int32)
    user3d = user.reshape(NW, NCH, CH)
    item3d = item.reshape(NW, NCH, CH)
    tugf = embed_user_GMF.T.reshape(-1)
    tigf = embed_item_GMF.T.reshape(-1)
    um, im, ug, ig = _sc_gather(user3d, item3d, tugf, tigf,
                                embed_user_MLP, embed_item_MLP)
    p1 = _tc_mlp(um, im,
                 W1[:M], W1[M:], b1.reshape(1, M),
                 W2, b2.reshape(1, M // 2),
                 W3, b3.reshape(1, F),
                 Wp[F:, 0].reshape(1, F), bp.reshape(1, 1))
    pred = _tc_gmf(ug, ig, p1.reshape(NW, NCH, CH),
                   Wp[:F, 0].reshape(1, F, 1, 1))
    return pred.reshape(B)
